# Initial kernel scaffold; baseline (speedup 1.0000x reference)
import jax
import jax.numpy as jnp
from jax.experimental import pallas as pl


def kernel(atom_rep, frag_rep, atom_Wih_f, atom_Whh_f, atom_bih_f, atom_bhh_f, atom_Wih_b, atom_Whh_b, atom_bih_b, atom_bhh_b, frag_Wih_f, frag_Whh_f, frag_bih_f, frag_bhh_f, frag_Wih_b, frag_Whh_b, frag_bih_b, frag_bhh_b, proj_W, proj_b):
    raise NotImplementedError("write your pallas kernel here")


# speedup vs baseline: 5.5924x; 5.5924x over previous
import functools

import jax
import jax.numpy as jnp
from jax.experimental import pallas as pl
from jax.experimental.pallas import tpu as pltpu


def _gru_scan_body(T, x_ref, wi_ref, wh_ref, bi_ref, bh_ref,
                   wb_ref, bib_ref, bhb_ref, hf_ref, hb_ref,
                   gr_s, gz_s, gn_s):
    c = pl.program_id(0)

    @pl.when(c == 0)
    def _init():
        hf_ref[...] = jnp.zeros_like(hf_ref)

    x = x_ref[...]
    dn = (((2,), (0,)), ((), ()))
    f32 = jnp.float32
    gr_s[...] = jax.lax.dot_general(x, wi_ref[0], dn,
                                    preferred_element_type=f32) + bi_ref[0]
    gz_s[...] = jax.lax.dot_general(x, wi_ref[1], dn,
                                    preferred_element_type=f32) + bi_ref[1]
    gn_s[...] = jax.lax.dot_general(x, wi_ref[2], dn,
                                    preferred_element_type=f32) + bi_ref[2]

    whr = wh_ref[0]
    whz = wh_ref[1]
    whn = wh_ref[2]
    bhr = bh_ref[0]
    bhz = bh_ref[1]
    bhn = bh_ref[2]

    def step(t, h):
        ghr = jnp.dot(h, whr, preferred_element_type=f32) + bhr
        ghz = jnp.dot(h, whz, preferred_element_type=f32) + bhz
        ghn = jnp.dot(h, whn, preferred_element_type=f32) + bhn
        r = jax.nn.sigmoid(gr_s[t] + ghr)
        z = jax.nn.sigmoid(gz_s[t] + ghz)
        n = jnp.tanh(gn_s[t] + r * ghn)
        return (1.0 - z) * n + z * h

    h = jax.lax.fori_loop(0, T, step, hf_ref[...])
    hf_ref[...] = h

    @pl.when(c == pl.num_programs(0) - 1)
    def _backward_last():
        xt = x[T - 1]
        gr = jnp.dot(xt, wb_ref[0], preferred_element_type=f32) + bib_ref[0]
        gz = jnp.dot(xt, wb_ref[1], preferred_element_type=f32) + bib_ref[1]
        gn = jnp.dot(xt, wb_ref[2], preferred_element_type=f32) + bib_ref[2]
        r = jax.nn.sigmoid(gr + bhb_ref[0])
        z = jax.nn.sigmoid(gz + bhb_ref[1])
        n = jnp.tanh(gn + r * bhb_ref[2])
        hb_ref[...] = (1.0 - z) * n


def _bigru_last(xs, wi, wh, bi, bh, wb, bib, bhb, T):
    S, B, H = xs.shape
    grid = S // T
    full3 = lambda c: (0, 0, 0)
    hf, hb = pl.pallas_call(
        functools.partial(_gru_scan_body, T),
        grid=(grid,),
        in_specs=[
            pl.BlockSpec((T, B, H), lambda c: (c, 0, 0)),
            pl.BlockSpec((3, H, H), full3),
            pl.BlockSpec((3, H, H), full3),
            pl.BlockSpec((3, 1, H), full3),
            pl.BlockSpec((3, 1, H), full3),
            pl.BlockSpec((3, H, H), full3),
            pl.BlockSpec((3, 1, H), full3),
            pl.BlockSpec((3, 1, H), full3),
        ],
        out_specs=[
            pl.BlockSpec((B, H), lambda c: (0, 0)),
            pl.BlockSpec((B, H), lambda c: (0, 0)),
        ],
        out_shape=[
            jax.ShapeDtypeStruct((B, H), jnp.float32),
            jax.ShapeDtypeStruct((B, H), jnp.float32),
        ],
        scratch_shapes=[
            pltpu.VMEM((T, B, H), jnp.float32),
            pltpu.VMEM((T, B, H), jnp.float32),
            pltpu.VMEM((T, B, H), jnp.float32),
        ],
    )(xs, wi, wh, bi, bh, wb, bib, bhb)
    return hf, hb


def _proj_body(ha_ref, ba_ref, hf_ref, bf_ref, p_ref, pb_ref, out_ref):
    f32 = jnp.float32
    acc = jnp.dot(ha_ref[...], p_ref[0], preferred_element_type=f32)
    acc = acc + jnp.dot(ba_ref[...], p_ref[1], preferred_element_type=f32)
    acc = acc + jnp.dot(hf_ref[...], p_ref[2], preferred_element_type=f32)
    acc = acc + jnp.dot(bf_ref[...], p_ref[3], preferred_element_type=f32)
    out_ref[...] = acc + pb_ref[...]


def _split_gates(W):
    H = W.shape[1]
    return jnp.swapaxes(W.reshape(3, H, H), 1, 2)


def kernel(atom_rep, frag_rep,
           atom_Wih_f, atom_Whh_f, atom_bih_f, atom_bhh_f,
           atom_Wih_b, atom_Whh_b, atom_bih_b, atom_bhh_b,
           frag_Wih_f, frag_Whh_f, frag_bih_f, frag_bhh_f,
           frag_Wih_b, frag_Whh_b, frag_bih_b, frag_bhh_b,
           proj_W, proj_b):
    B, S_atom, H = atom_rep.shape
    S_frag = frag_rep.shape[1]

    def prep(Wih, Whh, bih, bhh):
        return (_split_gates(Wih), _split_gates(Whh),
                bih.reshape(3, 1, H), bhh.reshape(3, 1, H))

    a_wi, a_wh, a_bi, a_bh = prep(atom_Wih_f, atom_Whh_f, atom_bih_f, atom_bhh_f)
    ab_wi, _, ab_bi, ab_bh = (_split_gates(atom_Wih_b), None,
                              atom_bih_b.reshape(3, 1, H),
                              atom_bhh_b.reshape(3, 1, H))
    f_wi, f_wh, f_bi, f_bh = prep(frag_Wih_f, frag_Whh_f, frag_bih_f, frag_bhh_f)
    fb_wi, _, fb_bi, fb_bh = (_split_gates(frag_Wih_b), None,
                              frag_bih_b.reshape(3, 1, H),
                              frag_bhh_b.reshape(3, 1, H))

    xa = jnp.swapaxes(atom_rep, 0, 1)
    xf = jnp.swapaxes(frag_rep, 0, 1)

    hA, bA = _bigru_last(xa, a_wi, a_wh, a_bi, a_bh, ab_wi, ab_bi, ab_bh, 16)
    hF, bF = _bigru_last(xf, f_wi, f_wh, f_bi, f_bh, fb_wi, fb_bi, fb_bh, 16)

    projT = proj_W.T.reshape(4, H, H)
    out = pl.pallas_call(
        _proj_body,
        in_specs=[pl.BlockSpec((B, H), lambda: (0, 0))] * 4
        + [pl.BlockSpec((4, H, H), lambda: (0, 0, 0)),
           pl.BlockSpec((1, H), lambda: (0, 0))],
        out_specs=pl.BlockSpec((B, H), lambda: (0, 0)),
        out_shape=jax.ShapeDtypeStruct((B, H), jnp.float32),
    )(hA, bA, hF, bF, projT, proj_b.reshape(1, H))
    return out
